# baseline (speedup 1.0000x reference)
import functools

import jax
import jax.numpy as jnp
from jax import lax
from jax.experimental import pallas as pl
from jax.experimental.pallas import tpu as pltpu
from jax.experimental.pallas import tpu_sc as plsc

_EMBED = 64
_NC = 2
_NS = 16
_NW = _NC * _NS
_ROWS_G = 640
_NBUF = 2


@functools.partial(jax.jit, static_argnames=("total",))
def _gather_rows(idx, table, *, total):
    per_w = idx.shape[2]
    groups = per_w // _ROWS_G
    assert groups % _NBUF == 0
    mesh = plsc.VectorSubcoreMesh(core_axis_name="c", subcore_axis_name="s")

    @functools.partial(
        pl.kernel,
        out_type=jax.ShapeDtypeStruct((total, _EMBED), jnp.float32),
        mesh=mesh,
        scratch_types=[
            pltpu.VMEM((1, per_w), jnp.int32),
            pltpu.VMEM((_NBUF, _ROWS_G, _EMBED), jnp.float32),
            [pltpu.SemaphoreType.DMA] * _NBUF,
            [pltpu.SemaphoreType.DMA] * _NBUF,
        ],
        compiler_params=pltpu.CompilerParams(use_tc_tiling_on_sc=False),
    )
    def body(idx_hbm, table_hbm, out_hbm, idx_v, rows, sem_g, sem_w):
        wid = lax.axis_index("s") * _NC + lax.axis_index("c")
        base = wid * per_w
        pltpu.sync_copy(idx_hbm.at[wid], idx_v)

        def gather(g, b):
            return pltpu.make_async_copy(
                table_hbm.at[idx_v.at[0, pl.ds(g * _ROWS_G, _ROWS_G)]],
                rows.at[b], sem_g[b])

        def write(g, b):
            return pltpu.make_async_copy(
                rows.at[b], out_hbm.at[pl.ds(base + g * _ROWS_G, _ROWS_G)],
                sem_w[b])

        for b in range(_NBUF):
            gather(b, b).start()

        @pl.loop(0, groups - _NBUF, step=_NBUF)
        def _(g0):
            for b in range(_NBUF):
                g = g0 + b
                gather(g, b).wait()
                write(g, b).start()
                write(g, b).wait()
                gather(g + _NBUF, b).start()

        for b in range(_NBUF):
            g = groups - _NBUF + b
            gather(g, b).wait()
            write(g, b).start()
        for b in range(_NBUF):
            g = groups - _NBUF + b
            write(g, b).wait()

    return body(idx, table)


def kernel(input, table):
    batch, hist = input.shape
    total = batch * hist
    idx = input.reshape(_NW, 1, total // _NW).astype(jnp.int32)
    out = _gather_rows(idx, table, total=total)
    return out.reshape(batch, hist, _EMBED)


# speedup vs baseline: 1.0006x; 1.0006x over previous
import functools

import jax
import jax.numpy as jnp
from jax import lax
from jax.experimental import pallas as pl
from jax.experimental.pallas import tpu as pltpu
from jax.experimental.pallas import tpu_sc as plsc

_EMBED = 64
_NC = 2
_NS = 16
_NW = _NC * _NS
_ROWS_G = 128
_NBUF = 8


@functools.partial(jax.jit, static_argnames=("total",))
def _gather_rows(idx, table, *, total):
    per_w = idx.shape[2]
    groups = per_w // _ROWS_G
    assert groups > _NBUF
    mesh = plsc.VectorSubcoreMesh(core_axis_name="c", subcore_axis_name="s")

    @functools.partial(
        pl.kernel,
        out_type=jax.ShapeDtypeStruct((total, _EMBED), jnp.float32),
        mesh=mesh,
        scratch_types=[
            pltpu.VMEM((1, per_w), jnp.int32),
            pltpu.VMEM((_NBUF, _ROWS_G, _EMBED), jnp.float32),
            [pltpu.SemaphoreType.DMA] * _NBUF,
            [pltpu.SemaphoreType.DMA] * _NBUF,
        ],
        compiler_params=pltpu.CompilerParams(use_tc_tiling_on_sc=False),
    )
    def body(idx_hbm, table_hbm, out_hbm, idx_v, rows, sem_g, sem_w):
        wid = lax.axis_index("s") * _NC + lax.axis_index("c")
        base = wid * per_w
        pltpu.sync_copy(idx_hbm.at[wid], idx_v)

        def gather(g):
            b = g % _NBUF
            return pltpu.make_async_copy(
                table_hbm.at[idx_v.at[0, pl.ds(g * _ROWS_G, _ROWS_G)]],
                rows.at[b], sem_g[b])

        def write(g):
            b = g % _NBUF
            return pltpu.make_async_copy(
                rows.at[b], out_hbm.at[pl.ds(base + g * _ROWS_G, _ROWS_G)],
                sem_w[b])

        for g in range(_NBUF):
            gather(g).start()
        for g in range(groups):
            gather(g).wait()
            write(g).start()
            if g >= 1 and g - 1 + _NBUF < groups:
                write(g - 1).wait()
                gather(g - 1 + _NBUF).start()
        for g in range(groups - _NBUF, groups):
            write(g).wait()

    return body(idx, table)


def kernel(input, table):
    batch, hist = input.shape
    total = batch * hist
    idx = input.reshape(_NW, 1, total // _NW).astype(jnp.int32)
    out = _gather_rows(idx, table, total=total)
    return out.reshape(batch, hist, _EMBED)
